# baseline (speedup 1.0000x reference)
import functools

import jax
import jax.numpy as jnp
from jax import lax
from jax.experimental import pallas as pl
from jax.experimental.pallas import tpu as pltpu
from jax.experimental.pallas import tpu_sc as plsc

_BATCH = 4
_SEQ = 2048
_D = 1024
_ROWS = _BATCH * _SEQ
_NW = 32
_RPW = _ROWS // _NW
_CH = 16
_NCH = _RPW // _CH
_CELEMS = _CH * _D
_NBUF = 2
_UNROLL = 8
_LANES = 16


def _chunk_add(xb, tb):
    @plsc.parallel_loop(0, _CELEMS, step=_LANES, unroll=_UNROLL)
    def _(i):
        plsc.addupdate(tb.at[pl.ds(i, _LANES)], xb[pl.ds(i, _LANES)])


@functools.partial(
    pl.kernel,
    mesh=plsc.VectorSubcoreMesh(core_axis_name="c", subcore_axis_name="s"),
    out_type=jax.ShapeDtypeStruct((_ROWS * _D,), jnp.float32),
    scratch_types=[
        pltpu.VMEM((_NBUF, _CELEMS), jnp.float32),
        pltpu.VMEM((_NBUF, _CELEMS), jnp.float32),
        pltpu.SemaphoreType.DMA((_NBUF,)),
        pltpu.SemaphoreType.DMA((_NBUF,)),
        pltpu.SemaphoreType.DMA((_NBUF,)),
    ],
)
def _sc_pos_add(x_hbm, tab_hbm, out_hbm, xbuf, tbuf, sem_x, sem_t, sem_o):
    cid = lax.axis_index("c")
    sid = lax.axis_index("s")
    wid = sid * 2 + cid
    row0 = wid * _RPW
    xoff0 = row0 * _D
    toff0 = (row0 % _SEQ) * _D

    def start_in(c):
        b = c % _NBUF
        xc = pltpu.async_copy(
            x_hbm.at[pl.ds(xoff0 + c * _CELEMS, _CELEMS)], xbuf.at[b], sem_x.at[b])
        tc = pltpu.async_copy(
            tab_hbm.at[pl.ds(toff0 + c * _CELEMS, _CELEMS)], tbuf.at[b], sem_t.at[b])
        return xc, tc

    in_flight = [None] * _NBUF
    out_flight = [None] * _NBUF

    in_flight[0] = start_in(0)
    for c in range(_NCH):
        b = c % _NBUF
        xc, tc = in_flight[b]
        xc.wait()
        tc.wait()
        if c + 1 < _NCH:
            nb = (c + 1) % _NBUF
            if out_flight[nb] is not None:
                out_flight[nb].wait()
            in_flight[nb] = start_in(c + 1)
        _chunk_add(xbuf.at[b], tbuf.at[b])
        out_flight[b] = pltpu.async_copy(
            tbuf.at[b], out_hbm.at[pl.ds(xoff0 + c * _CELEMS, _CELEMS)], sem_o.at[b])
    for b in range(_NBUF):
        if out_flight[b] is not None:
            out_flight[b].wait()


def kernel(x, pos_table):
    batch, seq_len, embed = x.shape
    out = _sc_pos_add(x.reshape(-1), pos_table[:seq_len].reshape(-1))
    return out.reshape(batch, seq_len, embed)
